# baseline (speedup 1.0000x reference)
import functools

import jax
import jax.numpy as jnp
from jax.experimental import pallas as pl

B, N, IN = 4, 2048, 1024
R = 256
E = 8
K = 2
OUT = 1024
TEMP = 1.0

TBLK = 512


def _fused_kernel(x_ref, wd_ref, wg_ref, wcat_ref, be_ref, wug_ref, s_ref,
                  out_ref):
    i = pl.program_id(0)
    xb = x_ref[...]
    hp = jax.lax.dot_general(xb, wd_ref[...], (((1,), (1,)), ((), ())),
                             preferred_element_type=jnp.float32)
    h = 0.5 * hp * (1.0 + jax.lax.erf(hp * 0.7071067811865476))

    lt = jax.lax.dot_general(wg_ref[...], h, (((1,), (1,)), ((), ())),
                             preferred_element_type=jnp.float32)
    lmax = jnp.max(lt, axis=0, keepdims=True)
    u = jnp.exp((lt - lmax) / TEMP)
    eidx = jax.lax.broadcasted_iota(jnp.int32, u.shape, 0)
    m1 = jnp.max(u, axis=0, keepdims=True)
    idx1 = jnp.min(jnp.where(u == m1, eidx, E), axis=0, keepdims=True)
    sel1 = eidx == idx1
    u2 = jnp.where(sel1, -jnp.inf, u)
    m2 = jnp.max(u2, axis=0, keepdims=True)
    idx2 = jnp.min(jnp.where(u2 == m2, eidx, E), axis=0, keepdims=True)
    sel2 = eidx == idx2
    denom = m1 + m2
    wt = (jnp.where(sel1, m1, 0.0) + jnp.where(sel2, m2, 0.0)) / denom

    col = jax.lax.broadcasted_iota(jnp.int32, u.shape, 1) + i * TBLK
    wt = jnp.where((col % N) >= 5, wt, 0.0)

    g = jax.lax.dot_general(h, wcat_ref[...], (((1,), (0,)), ((), ())),
                            preferred_element_type=jnp.float32)
    wrep = jax.lax.dot_general(wt, s_ref[...], (((0,), (0,)), ((), ())),
                               preferred_element_type=jnp.float32)
    acc = jax.lax.dot_general(wt, be_ref[...], (((0,), (0,)), ((), ())),
                              preferred_element_type=jnp.float32)
    gw = g * wrep
    for e in range(E):
        acc = acc + gw[:, e * R:(e + 1) * R]

    tok = h + acc
    out_ref[...] = jax.lax.dot_general(tok, wug_ref[...],
                                       (((1,), (1,)), ((), ())),
                                       preferred_element_type=jnp.float32)


@functools.partial(jax.jit, static_argnames=())
def kernel(x, Wd, Wg, We, be, Wu, gamma):
    xf = x.reshape(B * N, IN)
    wcat = jnp.transpose(We, (2, 0, 1)).reshape(R, E * R)
    wug = Wu * gamma[:, None]
    s = jnp.repeat(jnp.eye(E, dtype=jnp.float32), R, axis=1)
    grid = (B * N // TBLK,)
    out = pl.pallas_call(
        _fused_kernel,
        grid=grid,
        in_specs=[
            pl.BlockSpec((TBLK, IN), lambda i: (i, 0)),
            pl.BlockSpec((R, IN), lambda i: (0, 0)),
            pl.BlockSpec((E, R), lambda i: (0, 0)),
            pl.BlockSpec((R, E * R), lambda i: (0, 0)),
            pl.BlockSpec((E, R), lambda i: (0, 0)),
            pl.BlockSpec((OUT, R), lambda i: (0, 0)),
            pl.BlockSpec((E, E * R), lambda i: (0, 0)),
        ],
        out_specs=pl.BlockSpec((TBLK, OUT), lambda i: (i, 0)),
        out_shape=jax.ShapeDtypeStruct((B * N, OUT), jnp.float32),
    )(xf, Wd, Wg, wcat, be, wug, s)
    return out.reshape(B, N, OUT)


# speedup vs baseline: 1.1184x; 1.1184x over previous
import functools

import jax
import jax.numpy as jnp
from jax.experimental import pallas as pl

B, N, IN = 4, 2048, 1024
R = 256
E = 8
K = 2
OUT = 1024
TEMP = 1.0

TBLK = 512


def _fused_kernel(x_ref, wd_ref, wg_ref, wcat_ref, be_ref, wug_ref, s_ref,
                  out_ref):
    i = pl.program_id(0)
    xb = x_ref[...]
    hp = jax.lax.dot_general(xb, wd_ref[...], (((1,), (1,)), ((), ())),
                             preferred_element_type=jnp.float32)
    h = 0.5 * hp * (1.0 + jax.lax.erf(hp * 0.7071067811865476))

    lt = jax.lax.dot_general(wg_ref[...], h, (((1,), (1,)), ((), ())),
                             preferred_element_type=jnp.float32)
    lmax = jnp.max(lt, axis=0, keepdims=True)
    u = jnp.exp((lt - lmax) / TEMP)
    eidx = jax.lax.broadcasted_iota(jnp.int32, u.shape, 0)
    m1 = jnp.max(u, axis=0, keepdims=True)
    idx1 = jnp.min(jnp.where(u == m1, eidx, E), axis=0, keepdims=True)
    sel1 = eidx == idx1
    u2 = jnp.where(sel1, -jnp.inf, u)
    m2 = jnp.max(u2, axis=0, keepdims=True)
    idx2 = jnp.min(jnp.where(u2 == m2, eidx, E), axis=0, keepdims=True)
    sel2 = eidx == idx2
    denom = m1 + m2
    wt = (jnp.where(sel1, m1, 0.0) + jnp.where(sel2, m2, 0.0)) / denom

    col = jax.lax.broadcasted_iota(jnp.int32, u.shape, 1) + i * TBLK
    wt = jnp.where((col % N) >= 5, wt, 0.0)

    g = jax.lax.dot_general(h, wcat_ref[...], (((1,), (0,)), ((), ())),
                            preferred_element_type=jnp.float32)
    wtok = jax.lax.dot_general(wt, s_ref[...], (((0,), (0,)), ((), ())),
                               preferred_element_type=jnp.float32)
    acc = jax.lax.dot_general(wt, be_ref[...], (((0,), (0,)), ((), ())),
                              preferred_element_type=jnp.float32)
    for e in range(E):
        acc = acc + g[:, e * R:(e + 1) * R] * wtok[:, e:e + 1]

    tok = h + acc
    out_ref[...] = jax.lax.dot_general(tok, wug_ref[...],
                                       (((1,), (1,)), ((), ())),
                                       preferred_element_type=jnp.float32)


@functools.partial(jax.jit, static_argnames=())
def kernel(x, Wd, Wg, We, be, Wu, gamma):
    xf = x.reshape(B * N, IN)
    wcat = jnp.transpose(We, (2, 0, 1)).reshape(R, E * R)
    wug = Wu * gamma[:, None]
    s = jnp.eye(E, dtype=jnp.float32)
    grid = (B * N // TBLK,)
    out = pl.pallas_call(
        _fused_kernel,
        grid=grid,
        in_specs=[
            pl.BlockSpec((TBLK, IN), lambda i: (i, 0)),
            pl.BlockSpec((R, IN), lambda i: (0, 0)),
            pl.BlockSpec((E, R), lambda i: (0, 0)),
            pl.BlockSpec((R, E * R), lambda i: (0, 0)),
            pl.BlockSpec((E, R), lambda i: (0, 0)),
            pl.BlockSpec((OUT, R), lambda i: (0, 0)),
            pl.BlockSpec((E, E), lambda i: (0, 0)),
        ],
        out_specs=pl.BlockSpec((TBLK, OUT), lambda i: (i, 0)),
        out_shape=jax.ShapeDtypeStruct((B * N, OUT), jnp.float32),
    )(xf, Wd, Wg, wcat, be, wug, s)
    return out.reshape(B, N, OUT)
